# Initial kernel scaffold; baseline (speedup 1.0000x reference)
import jax
import jax.numpy as jnp
from jax.experimental import pallas as pl


def kernel(x, edge_index, edge_attr, batch, g1, mu1, sigma1, root1, b1, g2, mu2, sigma2, root2, b2, Wf1, bf1, Wf2, bf2):
    raise NotImplementedError("write your pallas kernel here")


# speedup vs baseline: 4.5743x; 4.5743x over previous
import functools

import jax
import jax.numpy as jnp
from jax import lax
from jax.experimental import pallas as pl
from jax.experimental.pallas import tpu as pltpu
from jax.experimental.pallas import tpu_sc as plsc

_N = 10000
_E = 320000
_K = 5
_NG = 64
_EPS = 1e-15
_NPAD = 10240
_HPAD = 5120
_CH = 80

_mesh = plsc.VectorSubcoreMesh(core_axis_name="c", subcore_axis_name="s")
_cparams = pltpu.CompilerParams(needs_layout_passes=False)


@functools.partial(
    pl.kernel,
    mesh=_mesh,
    compiler_params=_cparams,
    out_type=jax.ShapeDtypeStruct((2, _HPAD, 128), jnp.float32),
    scratch_types=[
        pltpu.VMEM((_N,), jnp.float32),
        pltpu.VMEM((4, 16), jnp.float32),
        pltpu.VMEM((10000,), jnp.int32),
        pltpu.VMEM((10000,), jnp.float32),
        pltpu.VMEM((10000,), jnp.float32),
        pltpu.VMEM((_CH,), jnp.int32),
        pltpu.VMEM((_CH,), jnp.int32),
        pltpu.VMEM((_CH, 128), jnp.float32),
        pltpu.VMEM_SHARED((_HPAD, 128), jnp.float32),
    ],
)
def _sc_pass1(src, dst, ea0, ea1, x, prm, out,
              x_v, prm_v, src_v, ea0_v, ea1_v, dstc, dstc2, pbuf, slab):
    c = lax.axis_index("c")
    t = lax.axis_index("s")
    lane = lax.iota(jnp.int32, 16)
    zeros16 = jnp.zeros((16,), jnp.float32)

    pltpu.sync_copy(x, x_v)
    pltpu.sync_copy(prm, prm_v)
    base = pl.multiple_of((c * 16 + t) * 10000, 8)
    pltpu.sync_copy(src.at[pl.ds(base, 10000)], src_v)
    pltpu.sync_copy(ea0.at[pl.ds(base, 10000)], ea0_v)
    pltpu.sync_copy(ea1.at[pl.ds(base, 10000)], ea1_v)

    def _z(i, _):
        for jr in range(8):
            pbuf[i, pl.ds(16 * jr, 16)] = zeros16
        return 0
    lax.fori_loop(0, _CH, _z, 0)
    rpt = _HPAD // 16
    for j in range(rpt // _CH):
        pltpu.sync_copy(pbuf, slab.at[pl.ds(t * rpt + j * _CH, _CH), :])
    plsc.subcore_barrier()

    mv0 = prm_v[0, :]
    mv1 = prm_v[1, :]
    sv0 = prm_v[2, :]
    sv1 = prm_v[3, :]
    cv0 = -0.5 / (_EPS + sv0 * sv0)
    cv1 = -0.5 / (_EPS + sv1 * sv1)
    cntv = jnp.where(lane == 5, 1.0, 0.0)
    is_g = lane < 5

    def _chunk(jj, _):
        off = pl.multiple_of(jj * _CH, 8)
        pltpu.sync_copy(dst.at[pl.ds(base + off, _CH)], dstc)
        for g in range(_CH // 16):
            d16 = dstc[pl.ds(16 * g, 16)]
            dstc2[pl.ds(16 * g, 16)] = lax.shift_right_logical(d16, 1)

        def _edge(e, _2):
            ev = jnp.full((16,), off + e, jnp.int32)
            sv = plsc.load_gather(src_v, [ev])
            xs = plsc.load_gather(x_v, [sv])
            e0 = plsc.load_gather(ea0_v, [ev])
            e1 = plsc.load_gather(ea1_v, [ev])
            dv = plsc.load_gather(dstc, [jnp.full((16,), e, jnp.int32)])
            even = (dv & 1) == 0
            d0 = e0 - mv0
            d1 = e1 - mv1
            gv = jnp.exp(d0 * d0 * cv0 + d1 * d1 * cv1)
            p16 = jnp.where(is_g, gv * xs, cntv)
            pbuf[e, pl.ds(0, 16)] = jnp.where(even, p16, zeros16)
            pbuf[e, pl.ds(64, 16)] = jnp.where(even, zeros16, p16)
            return 0

        lax.fori_loop(0, _CH, _edge, 0)
        pltpu.sync_copy(pbuf, slab.at[dstc2], add=True)
        return 0

    lax.fori_loop(0, 10000 // _CH, _chunk, 0)
    plsc.subcore_barrier()
    pltpu.sync_copy(slab.at[pl.ds(t * rpt, rpt), :],
                    out.at[c, pl.ds(t * rpt, rpt), :])


@functools.partial(
    pl.kernel,
    mesh=_mesh,
    compiler_params=_cparams,
    out_type=jax.ShapeDtypeStruct((2, _NPAD, 128), jnp.float32),
    scratch_types=[
        pltpu.VMEM((2, 32), jnp.float32),
        pltpu.VMEM((4000,), jnp.float32),
        pltpu.VMEM((4000,), jnp.float32),
        pltpu.VMEM((_CH,), jnp.int32),
        pltpu.VMEM((_CH,), jnp.int32),
        pltpu.VMEM((_CH, 128), jnp.float32),
        pltpu.VMEM((2 * _CH,), jnp.float32),
        pltpu.VMEM((_CH, 128), jnp.float32),
        pltpu.VMEM_SHARED((_NPAD, 128), jnp.float32),
        pltpu.SemaphoreType.DMA,
    ],
)
def _sc_pass2a(src, dst, ea0, ea1, h1, prm2, outa,
               prm_v, ea0_v, ea1_v, srcc, dstc, rows, gbuf, pbufa,
               slaba, sem):
    c = lax.axis_index("c")
    t = lax.axis_index("s")
    zeros16 = jnp.zeros((16,), jnp.float32)

    pltpu.sync_copy(prm2, prm_v)

    def _z(i, _):
        for jr in range(8):
            pbufa[i, pl.ds(16 * jr, 16)] = zeros16
        return 0
    lax.fori_loop(0, _CH, _z, 0)
    rpa = _NPAD // 16
    for j in range(rpa // _CH):
        pltpu.sync_copy(pbufa, slaba.at[pl.ds(t * rpa + j * _CH, _CH), :])
    plsc.subcore_barrier()

    pv_m = prm_v[c, pl.ds(0, 16)]
    pv_s = prm_v[c, pl.ds(16, 16)]
    pv_c = -0.5 / (_EPS + pv_s * pv_s)
    mks, cks = [], []
    for kk in range(2):
        mks.append((pv_m[2 * kk], pv_m[2 * kk + 1]))
        cks.append((pv_c[2 * kk], pv_c[2 * kk + 1]))

    tbase = t * 20000

    def _super(ss, _):
        sbase = pl.multiple_of(tbase + ss * 4000, 8)
        pltpu.sync_copy(ea0.at[pl.ds(sbase, 4000)], ea0_v)
        pltpu.sync_copy(ea1.at[pl.ds(sbase, 4000)], ea1_v)

        def _chunk(jj, _2):
            off = pl.multiple_of(jj * _CH, 8)
            pltpu.sync_copy(src.at[pl.ds(sbase + off, _CH)], srcc)
            pltpu.sync_copy(dst.at[pl.ds(sbase + off, _CH)], dstc)
            pltpu.async_copy(h1.at[srcc], rows, sem).wait()
            for g in range(_CH // 16):
                e0 = ea0_v[pl.ds(off + 16 * g, 16)]
                e1 = ea1_v[pl.ds(off + 16 * g, 16)]
                for kk in range(2):
                    d0 = e0 - mks[kk][0]
                    d1 = e1 - mks[kk][1]
                    gk = jnp.exp(d0 * d0 * cks[kk][0] + d1 * d1 * cks[kk][1])
                    gbuf[pl.ds(80 * kk + 16 * g, 16)] = gk

            def _edge(e, _3):
                s0 = plsc.load_gather(gbuf, [jnp.full((16,), e, jnp.int32)])
                s1 = plsc.load_gather(gbuf, [jnp.full((16,), 80 + e, jnp.int32)])
                for jr in range(4):
                    r = rows[e, pl.ds(16 * jr, 16)]
                    pbufa[e, pl.ds(16 * jr, 16)] = r * s0
                    pbufa[e, pl.ds(64 + 16 * jr, 16)] = r * s1
                return 0
            lax.fori_loop(0, _CH, _edge, 0)
            pltpu.sync_copy(pbufa, slaba.at[dstc], add=True)
            return 0

        lax.fori_loop(0, 50, _chunk, 0)
        return 0

    lax.fori_loop(0, 5, _super, 0)
    plsc.subcore_barrier()
    pltpu.sync_copy(slaba.at[pl.ds(t * rpa, rpa), :],
                    outa.at[c, pl.ds(t * rpa, rpa), :])


@functools.partial(
    pl.kernel,
    mesh=_mesh,
    compiler_params=_cparams,
    out_type=jax.ShapeDtypeStruct((2, _HPAD, 128), jnp.float32),
    scratch_types=[
        pltpu.VMEM((16,), jnp.float32),
        pltpu.VMEM((4000,), jnp.float32),
        pltpu.VMEM((4000,), jnp.float32),
        pltpu.VMEM((_CH,), jnp.int32),
        pltpu.VMEM((_CH,), jnp.int32),
        pltpu.VMEM((_CH,), jnp.int32),
        pltpu.VMEM((_CH, 128), jnp.float32),
        pltpu.VMEM((_CH,), jnp.float32),
        pltpu.VMEM((_CH, 128), jnp.float32),
        pltpu.VMEM_SHARED((_HPAD, 128), jnp.float32),
        pltpu.SemaphoreType.DMA,
    ],
)
def _sc_pass2b(src, dst, ea0, ea1, h1, prmb, outb,
               prm_v, ea0_v, ea1_v, srcc, dstc, dstc2, rows, gbuf, pbufb,
               slabb, sem):
    c = lax.axis_index("c")
    t = lax.axis_index("s")
    zeros16 = jnp.zeros((16,), jnp.float32)

    pltpu.sync_copy(prmb, prm_v)

    def _z(i, _):
        for jr in range(8):
            pbufb[i, pl.ds(16 * jr, 16)] = zeros16
        return 0
    lax.fori_loop(0, _CH, _z, 0)
    rpb = _HPAD // 16
    for j in range(rpb // _CH):
        pltpu.sync_copy(pbufb, slabb.at[pl.ds(t * rpb + j * _CH, _CH), :])
    plsc.subcore_barrier()

    pv = prm_v[...]
    pv_c = -0.5 / (_EPS + pv * pv)
    m0 = pv[0]
    m1 = pv[1]
    c0 = pv_c[2]
    c1 = pv_c[3]

    base = pl.multiple_of((c * 16 + t) * 10000, 8)

    def _super(ss, _):
        sbase = pl.multiple_of(base + ss * 4000, 8)
        pltpu.sync_copy(ea0.at[pl.ds(sbase, 4000)], ea0_v)
        pltpu.sync_copy(ea1.at[pl.ds(sbase, 4000)], ea1_v)

        def _chunk(jj, _2):
            off = pl.multiple_of(jj * _CH, 8)
            pltpu.sync_copy(src.at[pl.ds(sbase + off, _CH)], srcc)
            pltpu.sync_copy(dst.at[pl.ds(sbase + off, _CH)], dstc)
            pltpu.async_copy(h1.at[srcc], rows, sem).wait()
            for g in range(_CH // 16):
                e0 = ea0_v[pl.ds(off + 16 * g, 16)]
                e1 = ea1_v[pl.ds(off + 16 * g, 16)]
                d0 = e0 - m0
                d1 = e1 - m1
                gbuf[pl.ds(16 * g, 16)] = jnp.exp(d0 * d0 * c0 + d1 * d1 * c1)
                d16 = dstc[pl.ds(16 * g, 16)]
                dstc2[pl.ds(16 * g, 16)] = lax.shift_right_logical(d16, 1)

            def _edge(e, _3):
                s2 = plsc.load_gather(gbuf, [jnp.full((16,), e, jnp.int32)])
                dv = plsc.load_gather(dstc, [jnp.full((16,), e, jnp.int32)])
                even = (dv & 1) == 0
                for jr in range(4):
                    v = rows[e, pl.ds(16 * jr, 16)] * s2
                    pbufb[e, pl.ds(16 * jr, 16)] = jnp.where(even, v, zeros16)
                    pbufb[e, pl.ds(64 + 16 * jr, 16)] = \
                        jnp.where(even, zeros16, v)
                return 0
            lax.fori_loop(0, _CH, _edge, 0)
            pltpu.sync_copy(pbufb, slabb.at[dstc2], add=True)
            return 0

        lax.fori_loop(0, 50, _chunk, 0)
        return 0

    lax.fori_loop(0, 2, _super, 0)

    tb = pl.multiple_of(base + 8000, 8)
    pltpu.sync_copy(ea0.at[pl.ds(tb, 2000)], ea0_v.at[pl.ds(0, 2000)])
    pltpu.sync_copy(ea1.at[pl.ds(tb, 2000)], ea1_v.at[pl.ds(0, 2000)])

    def _chunk_t(jj, _2):
        off = pl.multiple_of(jj * _CH, 8)
        pltpu.sync_copy(src.at[pl.ds(tb + off, _CH)], srcc)
        pltpu.sync_copy(dst.at[pl.ds(tb + off, _CH)], dstc)
        pltpu.async_copy(h1.at[srcc], rows, sem).wait()
        for g in range(_CH // 16):
            e0 = ea0_v[pl.ds(off + 16 * g, 16)]
            e1 = ea1_v[pl.ds(off + 16 * g, 16)]
            d0 = e0 - m0
            d1 = e1 - m1
            gbuf[pl.ds(16 * g, 16)] = jnp.exp(d0 * d0 * c0 + d1 * d1 * c1)
            d16 = dstc[pl.ds(16 * g, 16)]
            dstc2[pl.ds(16 * g, 16)] = lax.shift_right_logical(d16, 1)

        def _edge(e, _3):
            s2 = plsc.load_gather(gbuf, [jnp.full((16,), e, jnp.int32)])
            dv = plsc.load_gather(dstc, [jnp.full((16,), e, jnp.int32)])
            even = (dv & 1) == 0
            for jr in range(4):
                v = rows[e, pl.ds(16 * jr, 16)] * s2
                pbufb[e, pl.ds(16 * jr, 16)] = jnp.where(even, v, zeros16)
                pbufb[e, pl.ds(64 + 16 * jr, 16)] = jnp.where(even, zeros16, v)
            return 0
        lax.fori_loop(0, _CH, _edge, 0)
        pltpu.sync_copy(pbufb, slabb.at[dstc2], add=True)
        return 0

    lax.fori_loop(0, 25, _chunk_t, 0)
    plsc.subcore_barrier()
    pltpu.sync_copy(slabb.at[pl.ds(t * rpb, rpb), :],
                    outb.at[c, pl.ds(t * rpb, rpb), :])


def _d1_body(s1_ref, x_ref, g1_ref, root1_ref, b1_ref, h1_ref, inv_ref):
    s = s1_ref[0] + s1_ref[1]
    a = s[:_N, 0:5]
    cnt = s[:_N, 5:6]
    inv = 1.0 / jnp.maximum(cnt, 1.0)
    h = jnp.dot(a, g1_ref[...], preferred_element_type=jnp.float32) * inv
    h = h + jnp.dot(x_ref[...], root1_ref[...],
                    preferred_element_type=jnp.float32) + b1_ref[...][None, :]
    h = jnp.maximum(h, 0.0)
    h1_ref[...] = jnp.concatenate(
        [h, jnp.zeros((_N, 64), jnp.float32)], axis=1)
    inv_ref[...] = inv


def _dense1(slab1r, x, G1, root1, b1):
    return pl.pallas_call(
        _d1_body,
        out_shape=[jax.ShapeDtypeStruct((_N, 128), jnp.float32),
                   jax.ShapeDtypeStruct((_N, 1), jnp.float32)],
    )(slab1r, x, G1, root1, b1)


def _d2_body(sa_ref, sb_ref, h1_ref, inv_ref, g2_ref, root2_ref, b2_ref,
             batch_ref, wf1_ref, bf1_ref, wf2_ref, bf2_ref, out_ref):
    k2 = sb_ref[0] + sb_ref[1]
    acat = jnp.concatenate([
        sa_ref[0][:_N, 0:64], sa_ref[0][:_N, 64:128],
        k2[:_N, :],
        sa_ref[1][:_N, 0:64], sa_ref[1][:_N, 64:128]], axis=1)
    agg = jnp.dot(acat, g2_ref[...],
                  preferred_element_type=jnp.float32) * inv_ref[...]
    h2 = agg + jnp.dot(h1_ref[...][:, 0:64], root2_ref[...],
                       preferred_element_type=jnp.float32) + b2_ref[...][None, :]
    h2 = jnp.maximum(h2, 0.0)
    gid = lax.broadcasted_iota(jnp.int32, (_NG, 1), 0)
    pm = (batch_ref[...] == gid).astype(jnp.float32)
    cg = jnp.sum(pm, axis=1, keepdims=True)
    p = jnp.dot(pm, h2, preferred_element_type=jnp.float32) / jnp.maximum(cg, 1.0)
    p = jnp.maximum(jnp.dot(p, wf1_ref[...], preferred_element_type=jnp.float32)
                    + bf1_ref[...][None, :], 0.0)
    lo = jnp.dot(p, wf2_ref[...], preferred_element_type=jnp.float32) \
        + bf2_ref[...][None, :]
    m = jnp.max(lo, axis=1, keepdims=True)
    lse = jnp.log(jnp.sum(jnp.exp(lo - m), axis=1, keepdims=True)) + m
    out_ref[...] = lo - lse


def _dense2(slaba, slabbr, h1, inv, G2cat, root2, b2, batch2d,
            Wf1, bf1, Wf2, bf2):
    return pl.pallas_call(
        _d2_body,
        out_shape=jax.ShapeDtypeStruct((_NG, 10), jnp.float32),
    )(slaba, slabbr, h1, inv, G2cat, root2, b2, batch2d, Wf1, bf1, Wf2, bf2)


def kernel(x, edge_index, edge_attr, batch, g1, mu1, sigma1, root1, b1,
           g2, mu2, sigma2, root2, b2, Wf1, bf1, Wf2, bf2):
    x1d = x.reshape(_N)
    batch2d = batch.reshape(1, _N)
    G1 = g1.reshape(_K, 64)
    G2cat = g2.reshape(64, _K, 128).transpose(1, 0, 2).reshape(_K * 64, 128)
    padz = jnp.zeros((11,), jnp.float32)
    pado = jnp.ones((11,), jnp.float32)
    pad10 = jnp.zeros((10,), jnp.float32)
    prm1 = jnp.stack([
        jnp.concatenate([mu1[:, 0], padz]),
        jnp.concatenate([mu1[:, 1], padz]),
        jnp.concatenate([sigma1[:, 0], pado]),
        jnp.concatenate([sigma1[:, 1], pado])])
    pad12z = jnp.zeros((12,), jnp.float32)
    pad12o = jnp.ones((12,), jnp.float32)
    sel0 = jnp.array([0, 1], dtype=jnp.int32)
    sel1 = jnp.array([3, 4], dtype=jnp.int32)
    prm2 = jnp.stack([
        jnp.concatenate([mu2[sel0].reshape(-1), pad12z,
                         sigma2[sel0].reshape(-1), pad12o]),
        jnp.concatenate([mu2[sel1].reshape(-1), pad12z,
                         sigma2[sel1].reshape(-1), pad12o])])
    prmb = jnp.concatenate([mu2[2], sigma2[2], jnp.ones((12,), jnp.float32)])
    src = edge_index[0]
    dst = edge_index[1]
    ea0 = edge_attr[:, 0]
    ea1 = edge_attr[:, 1]

    slab1 = _sc_pass1(src, dst, ea0, ea1, x1d, prm1)
    slab1r = slab1.reshape(2, _NPAD, 64)
    h1, inv = _dense1(slab1r, x, G1, root1, b1)
    slaba = _sc_pass2a(src, dst, ea0, ea1, h1, prm2)
    slabb = _sc_pass2b(src, dst, ea0, ea1, h1, prmb)
    slabbr = slabb.reshape(2, _NPAD, 64)
    return _dense2(slaba, slabbr, h1, inv, G2cat, root2, b2, batch2d,
                   Wf1, bf1, Wf2, bf2)


# speedup vs baseline: 6.2464x; 1.3656x over previous
import functools

import jax
import jax.numpy as jnp
from jax import lax
from jax.experimental import pallas as pl
from jax.experimental.pallas import tpu as pltpu
from jax.experimental.pallas import tpu_sc as plsc

_N = 10000
_E = 320000
_K = 5
_NG = 64
_EPS = 1e-15
_NPAD = 10240
_HPAD = 5120
_CH = 80

_mesh = plsc.VectorSubcoreMesh(core_axis_name="c", subcore_axis_name="s")
_cparams = pltpu.CompilerParams(needs_layout_passes=False)


@functools.partial(
    pl.kernel,
    mesh=_mesh,
    compiler_params=_cparams,
    out_type=jax.ShapeDtypeStruct((2, _HPAD, 128), jnp.float32),
    scratch_types=[
        pltpu.VMEM((_N,), jnp.float32),
        pltpu.VMEM((4, 16), jnp.float32),
        pltpu.VMEM((10000,), jnp.int32),
        pltpu.VMEM((10000,), jnp.float32),
        pltpu.VMEM((10000,), jnp.float32),
        pltpu.VMEM((_CH,), jnp.int32),
        pltpu.VMEM((_CH,), jnp.int32),
        pltpu.VMEM((_CH, 128), jnp.float32),
        pltpu.VMEM_SHARED((_HPAD, 128), jnp.float32),
    ],
)
def _sc_pass1(src, dst, ea0, ea1, x, prm, out,
              x_v, prm_v, src_v, ea0_v, ea1_v, dstc, dstc2, pbuf, slab):
    c = lax.axis_index("c")
    t = lax.axis_index("s")
    lane = lax.iota(jnp.int32, 16)
    zeros16 = jnp.zeros((16,), jnp.float32)

    pltpu.sync_copy(x, x_v)
    pltpu.sync_copy(prm, prm_v)
    base = pl.multiple_of((c * 16 + t) * 10000, 8)
    pltpu.sync_copy(src.at[pl.ds(base, 10000)], src_v)
    pltpu.sync_copy(ea0.at[pl.ds(base, 10000)], ea0_v)
    pltpu.sync_copy(ea1.at[pl.ds(base, 10000)], ea1_v)

    def _z(i, _):
        for jr in range(8):
            pbuf[i, pl.ds(16 * jr, 16)] = zeros16
        return 0
    lax.fori_loop(0, _CH, _z, 0)
    rpt = _HPAD // 16
    for j in range(rpt // _CH):
        pltpu.sync_copy(pbuf, slab.at[pl.ds(t * rpt + j * _CH, _CH), :])
    plsc.subcore_barrier()

    mv0 = prm_v[0, :]
    mv1 = prm_v[1, :]
    sv0 = prm_v[2, :]
    sv1 = prm_v[3, :]
    cv0 = -0.5 / (_EPS + sv0 * sv0)
    cv1 = -0.5 / (_EPS + sv1 * sv1)
    cntv = jnp.where(lane == 5, 1.0, 0.0)
    is_g = lane < 5

    def _chunk(jj, _):
        off = pl.multiple_of(jj * _CH, 8)
        pltpu.sync_copy(dst.at[pl.ds(base + off, _CH)], dstc)
        for g in range(_CH // 16):
            d16 = dstc[pl.ds(16 * g, 16)]
            dstc2[pl.ds(16 * g, 16)] = lax.shift_right_logical(d16, 1)

        def _edge(e, _2):
            ev = jnp.full((16,), off + e, jnp.int32)
            sv = plsc.load_gather(src_v, [ev])
            xs = plsc.load_gather(x_v, [sv])
            e0 = plsc.load_gather(ea0_v, [ev])
            e1 = plsc.load_gather(ea1_v, [ev])
            dv = plsc.load_gather(dstc, [jnp.full((16,), e, jnp.int32)])
            even = (dv & 1) == 0
            d0 = e0 - mv0
            d1 = e1 - mv1
            gv = jnp.exp(d0 * d0 * cv0 + d1 * d1 * cv1)
            p16 = jnp.where(is_g, gv * xs, cntv)
            pbuf[e, pl.ds(0, 16)] = jnp.where(even, p16, zeros16)
            pbuf[e, pl.ds(64, 16)] = jnp.where(even, zeros16, p16)
            return 0

        lax.fori_loop(0, _CH, _edge, 0)
        pltpu.sync_copy(pbuf, slab.at[dstc2], add=True)
        return 0

    lax.fori_loop(0, 10000 // _CH, _chunk, 0)
    plsc.subcore_barrier()
    pltpu.sync_copy(slab.at[pl.ds(t * rpt, rpt), :],
                    out.at[c, pl.ds(t * rpt, rpt), :])


@functools.partial(
    pl.kernel,
    mesh=_mesh,
    compiler_params=_cparams,
    out_type=jax.ShapeDtypeStruct((2, _NPAD, 128), jnp.float32),
    scratch_types=[
        pltpu.VMEM((2, 32), jnp.float32),
        pltpu.VMEM((800,), jnp.int32),
        pltpu.VMEM((800,), jnp.int32),
        pltpu.VMEM((800,), jnp.float32),
        pltpu.VMEM((800,), jnp.float32),
        pltpu.VMEM((_CH,), jnp.int32),
        pltpu.VMEM((_CH,), jnp.int32),
        pltpu.VMEM((_CH,), jnp.int32),
        pltpu.VMEM((_CH,), jnp.int32),
        pltpu.VMEM((_CH, 128), jnp.float32),
        pltpu.VMEM((_CH, 128), jnp.float32),
        pltpu.VMEM((2 * _CH,), jnp.float32),
        pltpu.VMEM((_CH, 128), jnp.float32),
        pltpu.VMEM((_CH, 128), jnp.float32),
        pltpu.VMEM_SHARED((_NPAD, 128), jnp.float32),
        pltpu.SemaphoreType.DMA,
        pltpu.SemaphoreType.DMA,
        pltpu.SemaphoreType.DMA,
        pltpu.SemaphoreType.DMA,
    ],
)
def _sc_pass2a(src, dst, ea0, ea1, h1, prm2, outa,
               prm_v, src_sv, dst_sv, ea0_sv, ea1_sv,
               srcca, srccb, dstca, dstcb, rowsa, rowsb, gbuf,
               pbufa, pbufb, slaba, sem_ra, sem_rb, sem_sa, sem_sb):
    c = lax.axis_index("c")
    t = lax.axis_index("s")
    zeros16 = jnp.zeros((16,), jnp.float32)

    pltpu.sync_copy(prm2, prm_v)

    def _z(i, _):
        for jr in range(8):
            pbufa[i, pl.ds(16 * jr, 16)] = zeros16
            pbufb[i, pl.ds(16 * jr, 16)] = zeros16
        return 0
    lax.fori_loop(0, _CH, _z, 0)
    rpa = _NPAD // 16
    for j in range(rpa // (2 * _CH)):
        pltpu.sync_copy(pbufa, slaba.at[pl.ds(t * rpa + 2 * j * _CH, _CH), :])
        pltpu.sync_copy(pbufb,
                        slaba.at[pl.ds(t * rpa + (2 * j + 1) * _CH, _CH), :])
    plsc.subcore_barrier()

    pv_m = prm_v[c, pl.ds(0, 16)]
    pv_s = prm_v[c, pl.ds(16, 16)]
    pv_c = -0.5 / (_EPS + pv_s * pv_s)
    mks, cks = [], []
    for kk in range(2):
        mks.append((pv_m[2 * kk], pv_m[2 * kk + 1]))
        cks.append((pv_c[2 * kk], pv_c[2 * kk + 1]))

    tbase = t * 20000

    def _load_super(s_idx):
        sb = pl.multiple_of(tbase + s_idx * 800, 8)
        pltpu.sync_copy(src.at[pl.ds(sb, 800)], src_sv)
        pltpu.sync_copy(dst.at[pl.ds(sb, 800)], dst_sv)
        pltpu.sync_copy(ea0.at[pl.ds(sb, 800)], ea0_sv)
        pltpu.sync_copy(ea1.at[pl.ds(sb, 800)], ea1_sv)

    def _fill_idx(buf, sv, loc):
        for g in range(5):
            buf[pl.ds(16 * g, 16)] = sv[pl.ds(loc + 16 * g, 16)]

    def _gauss(loc):
        for g in range(5):
            e0 = ea0_sv[pl.ds(loc + 16 * g, 16)]
            e1 = ea1_sv[pl.ds(loc + 16 * g, 16)]
            for kk in range(2):
                d0 = e0 - mks[kk][0]
                d1 = e1 - mks[kk][1]
                gk = jnp.exp(d0 * d0 * cks[kk][0] + d1 * d1 * cks[kk][1])
                gbuf[pl.ds(80 * kk + 16 * g, 16)] = gk

    def _payload(rows, pbuf):
        def _edge4(ii, _):
            for u in range(4):
                e = 4 * ii + u
                s0 = plsc.load_gather(gbuf, [jnp.full((16,), e, jnp.int32)])
                s1 = plsc.load_gather(gbuf,
                                      [jnp.full((16,), 80 + e, jnp.int32)])
                for jr in range(4):
                    r = rows[e, pl.ds(16 * jr, 16)]
                    pbuf[e, pl.ds(16 * jr, 16)] = r * s0
                    pbuf[e, pl.ds(64 + 16 * jr, 16)] = r * s1
            return 0
        lax.fori_loop(0, _CH // 4, _edge4, 0)

    _load_super(0)
    _fill_idx(srcca, src_sv, 0)
    pltpu.async_copy(h1.at[srcca], rowsa, sem_ra)

    def _body(i, _):
        m0 = 2 * i
        m1 = 2 * i + 1
        loc0 = lax.rem(m0, 10) * _CH
        loc1 = lax.rem(m1, 10) * _CH

        pltpu.make_async_copy(h1.at[srcca], rowsa, sem_ra).wait()
        _gauss(loc0)
        _fill_idx(srccb, src_sv, loc1)
        pltpu.async_copy(h1.at[srccb], rowsb, sem_rb)

        @pl.when(i > 0)
        def _w_sa():
            pltpu.make_async_copy(pbufa, slaba.at[dstca], sem_sa).wait()
        _fill_idx(dstca, dst_sv, loc0)
        _payload(rowsa, pbufa)
        pltpu.async_copy(pbufa, slaba.at[dstca], sem_sa, add=True)

        pltpu.make_async_copy(h1.at[srccb], rowsb, sem_rb).wait()
        _gauss(loc1)

        @pl.when(i > 0)
        def _w_sb():
            pltpu.make_async_copy(pbufb, slaba.at[dstcb], sem_sb).wait()
        _fill_idx(dstcb, dst_sv, loc1)

        @pl.when(jnp.logical_and(lax.rem(i, 5) == 4, i < 124))
        def _ns():
            _load_super((m1 + 1) // 10)

        @pl.when(i < 124)
        def _pf():
            loc2 = lax.rem(m1 + 1, 10) * _CH
            _fill_idx(srcca, src_sv, loc2)
            pltpu.async_copy(h1.at[srcca], rowsa, sem_ra)

        _payload(rowsb, pbufb)
        pltpu.async_copy(pbufb, slaba.at[dstcb], sem_sb, add=True)
        return 0

    lax.fori_loop(0, 125, _body, 0)
    pltpu.make_async_copy(pbufa, slaba.at[dstca], sem_sa).wait()
    pltpu.make_async_copy(pbufb, slaba.at[dstcb], sem_sb).wait()
    plsc.subcore_barrier()
    pltpu.sync_copy(slaba.at[pl.ds(t * rpa, rpa), :],
                    outa.at[c, pl.ds(t * rpa, rpa), :])


@functools.partial(
    pl.kernel,
    mesh=_mesh,
    compiler_params=_cparams,
    out_type=jax.ShapeDtypeStruct((2, _HPAD, 128), jnp.float32),
    scratch_types=[
        pltpu.VMEM((16,), jnp.float32),
        pltpu.VMEM((4000,), jnp.float32),
        pltpu.VMEM((4000,), jnp.float32),
        pltpu.VMEM((_CH,), jnp.int32),
        pltpu.VMEM((_CH,), jnp.int32),
        pltpu.VMEM((_CH,), jnp.int32),
        pltpu.VMEM((_CH, 128), jnp.float32),
        pltpu.VMEM((_CH,), jnp.float32),
        pltpu.VMEM((_CH, 128), jnp.float32),
        pltpu.VMEM_SHARED((_HPAD, 128), jnp.float32),
        pltpu.SemaphoreType.DMA,
    ],
)
def _sc_pass2b(src, dst, ea0, ea1, h1, prmb, outb,
               prm_v, ea0_v, ea1_v, srcc, dstc, dstc2, rows, gbuf, pbufb,
               slabb, sem):
    c = lax.axis_index("c")
    t = lax.axis_index("s")
    zeros16 = jnp.zeros((16,), jnp.float32)

    pltpu.sync_copy(prmb, prm_v)

    def _z(i, _):
        for jr in range(8):
            pbufb[i, pl.ds(16 * jr, 16)] = zeros16
        return 0
    lax.fori_loop(0, _CH, _z, 0)
    rpb = _HPAD // 16
    for j in range(rpb // _CH):
        pltpu.sync_copy(pbufb, slabb.at[pl.ds(t * rpb + j * _CH, _CH), :])
    plsc.subcore_barrier()

    pv = prm_v[...]
    pv_c = -0.5 / (_EPS + pv * pv)
    m0 = pv[0]
    m1 = pv[1]
    c0 = pv_c[2]
    c1 = pv_c[3]

    base = pl.multiple_of((c * 16 + t) * 10000, 8)

    def _super(ss, _):
        sbase = pl.multiple_of(base + ss * 4000, 8)
        pltpu.sync_copy(ea0.at[pl.ds(sbase, 4000)], ea0_v)
        pltpu.sync_copy(ea1.at[pl.ds(sbase, 4000)], ea1_v)

        def _chunk(jj, _2):
            off = pl.multiple_of(jj * _CH, 8)
            pltpu.sync_copy(src.at[pl.ds(sbase + off, _CH)], srcc)
            pltpu.sync_copy(dst.at[pl.ds(sbase + off, _CH)], dstc)
            pltpu.async_copy(h1.at[srcc], rows, sem).wait()
            for g in range(_CH // 16):
                e0 = ea0_v[pl.ds(off + 16 * g, 16)]
                e1 = ea1_v[pl.ds(off + 16 * g, 16)]
                d0 = e0 - m0
                d1 = e1 - m1
                gbuf[pl.ds(16 * g, 16)] = jnp.exp(d0 * d0 * c0 + d1 * d1 * c1)
                d16 = dstc[pl.ds(16 * g, 16)]
                dstc2[pl.ds(16 * g, 16)] = lax.shift_right_logical(d16, 1)

            def _edge(e, _3):
                s2 = plsc.load_gather(gbuf, [jnp.full((16,), e, jnp.int32)])
                dv = plsc.load_gather(dstc, [jnp.full((16,), e, jnp.int32)])
                even = (dv & 1) == 0
                for jr in range(4):
                    v = rows[e, pl.ds(16 * jr, 16)] * s2
                    pbufb[e, pl.ds(16 * jr, 16)] = jnp.where(even, v, zeros16)
                    pbufb[e, pl.ds(64 + 16 * jr, 16)] = \
                        jnp.where(even, zeros16, v)
                return 0
            lax.fori_loop(0, _CH, _edge, 0)
            pltpu.sync_copy(pbufb, slabb.at[dstc2], add=True)
            return 0

        lax.fori_loop(0, 50, _chunk, 0)
        return 0

    lax.fori_loop(0, 2, _super, 0)

    tb = pl.multiple_of(base + 8000, 8)
    pltpu.sync_copy(ea0.at[pl.ds(tb, 2000)], ea0_v.at[pl.ds(0, 2000)])
    pltpu.sync_copy(ea1.at[pl.ds(tb, 2000)], ea1_v.at[pl.ds(0, 2000)])

    def _chunk_t(jj, _2):
        off = pl.multiple_of(jj * _CH, 8)
        pltpu.sync_copy(src.at[pl.ds(tb + off, _CH)], srcc)
        pltpu.sync_copy(dst.at[pl.ds(tb + off, _CH)], dstc)
        pltpu.async_copy(h1.at[srcc], rows, sem).wait()
        for g in range(_CH // 16):
            e0 = ea0_v[pl.ds(off + 16 * g, 16)]
            e1 = ea1_v[pl.ds(off + 16 * g, 16)]
            d0 = e0 - m0
            d1 = e1 - m1
            gbuf[pl.ds(16 * g, 16)] = jnp.exp(d0 * d0 * c0 + d1 * d1 * c1)
            d16 = dstc[pl.ds(16 * g, 16)]
            dstc2[pl.ds(16 * g, 16)] = lax.shift_right_logical(d16, 1)

        def _edge(e, _3):
            s2 = plsc.load_gather(gbuf, [jnp.full((16,), e, jnp.int32)])
            dv = plsc.load_gather(dstc, [jnp.full((16,), e, jnp.int32)])
            even = (dv & 1) == 0
            for jr in range(4):
                v = rows[e, pl.ds(16 * jr, 16)] * s2
                pbufb[e, pl.ds(16 * jr, 16)] = jnp.where(even, v, zeros16)
                pbufb[e, pl.ds(64 + 16 * jr, 16)] = jnp.where(even, zeros16, v)
            return 0
        lax.fori_loop(0, _CH, _edge, 0)
        pltpu.sync_copy(pbufb, slabb.at[dstc2], add=True)
        return 0

    lax.fori_loop(0, 25, _chunk_t, 0)
    plsc.subcore_barrier()
    pltpu.sync_copy(slabb.at[pl.ds(t * rpb, rpb), :],
                    outb.at[c, pl.ds(t * rpb, rpb), :])


def _d1_body(s1_ref, x_ref, g1_ref, root1_ref, b1_ref, h1_ref, inv_ref):
    s = s1_ref[0] + s1_ref[1]
    a = s[:_N, 0:5]
    cnt = s[:_N, 5:6]
    inv = 1.0 / jnp.maximum(cnt, 1.0)
    h = jnp.dot(a, g1_ref[...], preferred_element_type=jnp.float32) * inv
    h = h + jnp.dot(x_ref[...], root1_ref[...],
                    preferred_element_type=jnp.float32) + b1_ref[...][None, :]
    h = jnp.maximum(h, 0.0)
    h1_ref[...] = jnp.concatenate(
        [h, jnp.zeros((_N, 64), jnp.float32)], axis=1)
    inv_ref[...] = inv


def _dense1(slab1r, x, G1, root1, b1):
    return pl.pallas_call(
        _d1_body,
        out_shape=[jax.ShapeDtypeStruct((_N, 128), jnp.float32),
                   jax.ShapeDtypeStruct((_N, 1), jnp.float32)],
    )(slab1r, x, G1, root1, b1)


def _d2_body(sa_ref, sb_ref, h1_ref, inv_ref, g2_ref, root2_ref, b2_ref,
             batch_ref, wf1_ref, bf1_ref, wf2_ref, bf2_ref, out_ref):
    k2 = sb_ref[0] + sb_ref[1]
    acat = jnp.concatenate([
        sa_ref[0][:_N, 0:64], sa_ref[0][:_N, 64:128],
        k2[:_N, :],
        sa_ref[1][:_N, 0:64], sa_ref[1][:_N, 64:128]], axis=1)
    agg = jnp.dot(acat, g2_ref[...],
                  preferred_element_type=jnp.float32) * inv_ref[...]
    h2 = agg + jnp.dot(h1_ref[...][:, 0:64], root2_ref[...],
                       preferred_element_type=jnp.float32) + b2_ref[...][None, :]
    h2 = jnp.maximum(h2, 0.0)
    gid = lax.broadcasted_iota(jnp.int32, (_NG, 1), 0)
    pm = (batch_ref[...] == gid).astype(jnp.float32)
    cg = jnp.sum(pm, axis=1, keepdims=True)
    p = jnp.dot(pm, h2, preferred_element_type=jnp.float32) / jnp.maximum(cg, 1.0)
    p = jnp.maximum(jnp.dot(p, wf1_ref[...], preferred_element_type=jnp.float32)
                    + bf1_ref[...][None, :], 0.0)
    lo = jnp.dot(p, wf2_ref[...], preferred_element_type=jnp.float32) \
        + bf2_ref[...][None, :]
    m = jnp.max(lo, axis=1, keepdims=True)
    lse = jnp.log(jnp.sum(jnp.exp(lo - m), axis=1, keepdims=True)) + m
    out_ref[...] = lo - lse


def _dense2(slaba, slabbr, h1, inv, G2cat, root2, b2, batch2d,
            Wf1, bf1, Wf2, bf2):
    return pl.pallas_call(
        _d2_body,
        out_shape=jax.ShapeDtypeStruct((_NG, 10), jnp.float32),
    )(slaba, slabbr, h1, inv, G2cat, root2, b2, batch2d, Wf1, bf1, Wf2, bf2)


def kernel(x, edge_index, edge_attr, batch, g1, mu1, sigma1, root1, b1,
           g2, mu2, sigma2, root2, b2, Wf1, bf1, Wf2, bf2):
    x1d = x.reshape(_N)
    batch2d = batch.reshape(1, _N)
    G1 = g1.reshape(_K, 64)
    G2cat = g2.reshape(64, _K, 128).transpose(1, 0, 2).reshape(_K * 64, 128)
    padz = jnp.zeros((11,), jnp.float32)
    pado = jnp.ones((11,), jnp.float32)
    pad10 = jnp.zeros((10,), jnp.float32)
    prm1 = jnp.stack([
        jnp.concatenate([mu1[:, 0], padz]),
        jnp.concatenate([mu1[:, 1], padz]),
        jnp.concatenate([sigma1[:, 0], pado]),
        jnp.concatenate([sigma1[:, 1], pado])])
    pad12z = jnp.zeros((12,), jnp.float32)
    pad12o = jnp.ones((12,), jnp.float32)
    sel0 = jnp.array([0, 1], dtype=jnp.int32)
    sel1 = jnp.array([3, 4], dtype=jnp.int32)
    prm2 = jnp.stack([
        jnp.concatenate([mu2[sel0].reshape(-1), pad12z,
                         sigma2[sel0].reshape(-1), pad12o]),
        jnp.concatenate([mu2[sel1].reshape(-1), pad12z,
                         sigma2[sel1].reshape(-1), pad12o])])
    prmb = jnp.concatenate([mu2[2], sigma2[2], jnp.ones((12,), jnp.float32)])
    src = edge_index[0]
    dst = edge_index[1]
    ea0 = edge_attr[:, 0]
    ea1 = edge_attr[:, 1]

    slab1 = _sc_pass1(src, dst, ea0, ea1, x1d, prm1)
    slab1r = slab1.reshape(2, _NPAD, 64)
    h1, inv = _dense1(slab1r, x, G1, root1, b1)
    slaba = _sc_pass2a(src, dst, ea0, ea1, h1, prm2)
    slabb = _sc_pass2b(src, dst, ea0, ea1, h1, prmb)
    slabbr = slabb.reshape(2, _NPAD, 64)
    return _dense2(slaba, slabbr, h1, inv, G2cat, root2, b2, batch2d,
                   Wf1, bf1, Wf2, bf2)


# speedup vs baseline: 8.4073x; 1.3459x over previous
import functools

import jax
import jax.numpy as jnp
from jax import lax
from jax.experimental import pallas as pl
from jax.experimental.pallas import tpu as pltpu
from jax.experimental.pallas import tpu_sc as plsc

_N = 10000
_E = 320000
_K = 5
_NG = 64
_EPS = 1e-15
_NPAD = 10240
_HPAD = 5120
_CH = 80

_mesh = plsc.VectorSubcoreMesh(core_axis_name="c", subcore_axis_name="s")
_cparams = pltpu.CompilerParams(needs_layout_passes=False)


@functools.partial(
    pl.kernel,
    mesh=_mesh,
    compiler_params=_cparams,
    out_type=jax.ShapeDtypeStruct((2, _HPAD, 128), jnp.float32),
    scratch_types=[
        pltpu.VMEM((_N,), jnp.float32),
        pltpu.VMEM((4, 16), jnp.float32),
        pltpu.VMEM((10000,), jnp.int32),
        pltpu.VMEM((10000,), jnp.int32),
        pltpu.VMEM((10000,), jnp.float32),
        pltpu.VMEM((10000,), jnp.float32),
        pltpu.VMEM((_CH,), jnp.int32),
        pltpu.VMEM((_CH,), jnp.int32),
        pltpu.VMEM((_CH, 128), jnp.float32),
        pltpu.VMEM((_CH, 128), jnp.float32),
        pltpu.VMEM_SHARED((_HPAD, 128), jnp.float32),
        pltpu.SemaphoreType.DMA,
        pltpu.SemaphoreType.DMA,
    ],
)
def _sc_pass1(src, dst, ea0, ea1, x, prm, out,
              x_v, prm_v, src_v, dst_v, ea0_v, ea1_v, dstc2a, dstc2b,
              pbufa, pbufb, slab, sem_sa, sem_sb):
    c = lax.axis_index("c")
    t = lax.axis_index("s")
    lane = lax.iota(jnp.int32, 16)
    zeros16 = jnp.zeros((16,), jnp.float32)

    pltpu.sync_copy(x, x_v)
    pltpu.sync_copy(prm, prm_v)
    base = pl.multiple_of((c * 16 + t) * 10000, 8)
    pltpu.sync_copy(src.at[pl.ds(base, 10000)], src_v)
    pltpu.sync_copy(dst.at[pl.ds(base, 10000)], dst_v)
    pltpu.sync_copy(ea0.at[pl.ds(base, 10000)], ea0_v)
    pltpu.sync_copy(ea1.at[pl.ds(base, 10000)], ea1_v)

    def _z(i, _):
        for jr in range(8):
            pbufa[i, pl.ds(16 * jr, 16)] = zeros16
            pbufb[i, pl.ds(16 * jr, 16)] = zeros16
        return 0
    lax.fori_loop(0, _CH, _z, 0)
    rpt = _HPAD // 16
    for j in range(rpt // (2 * _CH)):
        pltpu.sync_copy(pbufa, slab.at[pl.ds(t * rpt + 2 * j * _CH, _CH), :])
        pltpu.sync_copy(pbufb,
                        slab.at[pl.ds(t * rpt + (2 * j + 1) * _CH, _CH), :])
    plsc.subcore_barrier()

    mv0 = prm_v[0, :]
    mv1 = prm_v[1, :]
    sv0 = prm_v[2, :]
    sv1 = prm_v[3, :]
    cv0 = -0.5 / (_EPS + sv0 * sv0)
    cv1 = -0.5 / (_EPS + sv1 * sv1)
    cntv = jnp.where(lane == 5, 1.0, 0.0)
    is_g = lane < 5

    def _fill_dst2(buf, loc):
        for g in range(5):
            d16 = dst_v[pl.ds(loc + 16 * g, 16)]
            buf[pl.ds(16 * g, 16)] = lax.shift_right_logical(d16, 1)

    def _payload(loc, pbuf):
        def _edge4(ii, _2):
            for u in range(4):
                e = 4 * ii + u
                ev = jnp.full((16,), loc + e, jnp.int32)
                sv = plsc.load_gather(src_v, [ev])
                xs = plsc.load_gather(x_v, [sv])
                e0 = plsc.load_gather(ea0_v, [ev])
                e1 = plsc.load_gather(ea1_v, [ev])
                dv = plsc.load_gather(dst_v, [ev])
                even = (dv & 1) == 0
                d0 = e0 - mv0
                d1 = e1 - mv1
                gv = jnp.exp(d0 * d0 * cv0 + d1 * d1 * cv1)
                p16 = jnp.where(is_g, gv * xs, cntv)
                pbuf[e, pl.ds(0, 16)] = jnp.where(even, p16, zeros16)
                pbuf[e, pl.ds(64, 16)] = jnp.where(even, zeros16, p16)
            return 0
        lax.fori_loop(0, _CH // 4, _edge4, 0)

    def _body(i, _):
        loc0 = 2 * i * _CH
        loc1 = loc0 + _CH

        @pl.when(i > 0)
        def _w_sa():
            pltpu.make_async_copy(pbufa, slab.at[dstc2a], sem_sa).wait()
        _fill_dst2(dstc2a, loc0)
        _payload(loc0, pbufa)
        pltpu.async_copy(pbufa, slab.at[dstc2a], sem_sa, add=True)

        @pl.when(i > 0)
        def _w_sb():
            pltpu.make_async_copy(pbufb, slab.at[dstc2b], sem_sb).wait()
        _fill_dst2(dstc2b, loc1)
        _payload(loc1, pbufb)
        pltpu.async_copy(pbufb, slab.at[dstc2b], sem_sb, add=True)
        return 0

    lax.fori_loop(0, 62, _body, 0)

    loct = 124 * _CH
    pltpu.make_async_copy(pbufa, slab.at[dstc2a], sem_sa).wait()
    _fill_dst2(dstc2a, loct)
    _payload(loct, pbufa)
    pltpu.async_copy(pbufa, slab.at[dstc2a], sem_sa, add=True)

    pltpu.make_async_copy(pbufa, slab.at[dstc2a], sem_sa).wait()
    pltpu.make_async_copy(pbufb, slab.at[dstc2b], sem_sb).wait()
    plsc.subcore_barrier()
    pltpu.sync_copy(slab.at[pl.ds(t * rpt, rpt), :],
                    out.at[c, pl.ds(t * rpt, rpt), :])


@functools.partial(
    pl.kernel,
    mesh=_mesh,
    compiler_params=_cparams,
    out_type=jax.ShapeDtypeStruct((2, _NPAD, 128), jnp.float32),
    scratch_types=[
        pltpu.VMEM((2, 32), jnp.float32),
        pltpu.VMEM((800,), jnp.int32),
        pltpu.VMEM((800,), jnp.int32),
        pltpu.VMEM((800,), jnp.float32),
        pltpu.VMEM((800,), jnp.float32),
        pltpu.VMEM((_CH,), jnp.int32),
        pltpu.VMEM((_CH,), jnp.int32),
        pltpu.VMEM((_CH,), jnp.int32),
        pltpu.VMEM((_CH,), jnp.int32),
        pltpu.VMEM((_CH, 128), jnp.float32),
        pltpu.VMEM((_CH, 128), jnp.float32),
        pltpu.VMEM((2 * _CH,), jnp.float32),
        pltpu.VMEM((_CH, 128), jnp.float32),
        pltpu.VMEM((_CH, 128), jnp.float32),
        pltpu.VMEM_SHARED((_NPAD, 128), jnp.float32),
        pltpu.SemaphoreType.DMA,
        pltpu.SemaphoreType.DMA,
        pltpu.SemaphoreType.DMA,
        pltpu.SemaphoreType.DMA,
    ],
)
def _sc_pass2a(src, dst, ea0, ea1, h1, prm2, outa,
               prm_v, src_sv, dst_sv, ea0_sv, ea1_sv,
               srcca, srccb, dstca, dstcb, rowsa, rowsb, gbuf,
               pbufa, pbufb, slaba, sem_ra, sem_rb, sem_sa, sem_sb):
    c = lax.axis_index("c")
    t = lax.axis_index("s")
    zeros16 = jnp.zeros((16,), jnp.float32)

    pltpu.sync_copy(prm2, prm_v)

    def _z(i, _):
        for jr in range(8):
            pbufa[i, pl.ds(16 * jr, 16)] = zeros16
            pbufb[i, pl.ds(16 * jr, 16)] = zeros16
        return 0
    lax.fori_loop(0, _CH, _z, 0)
    rpa = _NPAD // 16
    for j in range(rpa // (2 * _CH)):
        pltpu.sync_copy(pbufa, slaba.at[pl.ds(t * rpa + 2 * j * _CH, _CH), :])
        pltpu.sync_copy(pbufb,
                        slaba.at[pl.ds(t * rpa + (2 * j + 1) * _CH, _CH), :])
    plsc.subcore_barrier()

    pv_m = prm_v[c, pl.ds(0, 16)]
    pv_s = prm_v[c, pl.ds(16, 16)]
    pv_c = -0.5 / (_EPS + pv_s * pv_s)
    mks, cks = [], []
    for kk in range(2):
        mks.append((pv_m[2 * kk], pv_m[2 * kk + 1]))
        cks.append((pv_c[2 * kk], pv_c[2 * kk + 1]))

    tbase = t * 20000

    def _load_super(s_idx):
        sb = pl.multiple_of(tbase + s_idx * 800, 8)
        pltpu.sync_copy(src.at[pl.ds(sb, 800)], src_sv)
        pltpu.sync_copy(dst.at[pl.ds(sb, 800)], dst_sv)
        pltpu.sync_copy(ea0.at[pl.ds(sb, 800)], ea0_sv)
        pltpu.sync_copy(ea1.at[pl.ds(sb, 800)], ea1_sv)

    def _fill_idx(buf, sv, loc):
        for g in range(5):
            buf[pl.ds(16 * g, 16)] = sv[pl.ds(loc + 16 * g, 16)]

    def _gauss(loc):
        for g in range(5):
            e0 = ea0_sv[pl.ds(loc + 16 * g, 16)]
            e1 = ea1_sv[pl.ds(loc + 16 * g, 16)]
            for kk in range(2):
                d0 = e0 - mks[kk][0]
                d1 = e1 - mks[kk][1]
                gk = jnp.exp(d0 * d0 * cks[kk][0] + d1 * d1 * cks[kk][1])
                gbuf[pl.ds(80 * kk + 16 * g, 16)] = gk

    def _payload(rows, pbuf):
        def _edge4(ii, _):
            for u in range(4):
                e = 4 * ii + u
                s0 = plsc.load_gather(gbuf, [jnp.full((16,), e, jnp.int32)])
                s1 = plsc.load_gather(gbuf,
                                      [jnp.full((16,), 80 + e, jnp.int32)])
                for jr in range(4):
                    r = rows[e, pl.ds(16 * jr, 16)]
                    pbuf[e, pl.ds(16 * jr, 16)] = r * s0
                    pbuf[e, pl.ds(64 + 16 * jr, 16)] = r * s1
            return 0
        lax.fori_loop(0, _CH // 4, _edge4, 0)

    _load_super(0)
    _fill_idx(srcca, src_sv, 0)
    pltpu.async_copy(h1.at[srcca], rowsa, sem_ra)

    def _body(i, _):
        m0 = 2 * i
        m1 = 2 * i + 1
        loc0 = lax.rem(m0, 10) * _CH
        loc1 = lax.rem(m1, 10) * _CH

        pltpu.make_async_copy(h1.at[srcca], rowsa, sem_ra).wait()
        _gauss(loc0)
        _fill_idx(srccb, src_sv, loc1)
        pltpu.async_copy(h1.at[srccb], rowsb, sem_rb)

        @pl.when(i > 0)
        def _w_sa():
            pltpu.make_async_copy(pbufa, slaba.at[dstca], sem_sa).wait()
        _fill_idx(dstca, dst_sv, loc0)
        _payload(rowsa, pbufa)
        pltpu.async_copy(pbufa, slaba.at[dstca], sem_sa, add=True)

        pltpu.make_async_copy(h1.at[srccb], rowsb, sem_rb).wait()
        _gauss(loc1)

        @pl.when(i > 0)
        def _w_sb():
            pltpu.make_async_copy(pbufb, slaba.at[dstcb], sem_sb).wait()
        _fill_idx(dstcb, dst_sv, loc1)

        @pl.when(jnp.logical_and(lax.rem(i, 5) == 4, i < 124))
        def _ns():
            _load_super((m1 + 1) // 10)

        @pl.when(i < 124)
        def _pf():
            loc2 = lax.rem(m1 + 1, 10) * _CH
            _fill_idx(srcca, src_sv, loc2)
            pltpu.async_copy(h1.at[srcca], rowsa, sem_ra)

        _payload(rowsb, pbufb)
        pltpu.async_copy(pbufb, slaba.at[dstcb], sem_sb, add=True)
        return 0

    lax.fori_loop(0, 125, _body, 0)
    pltpu.make_async_copy(pbufa, slaba.at[dstca], sem_sa).wait()
    pltpu.make_async_copy(pbufb, slaba.at[dstcb], sem_sb).wait()
    plsc.subcore_barrier()
    pltpu.sync_copy(slaba.at[pl.ds(t * rpa, rpa), :],
                    outa.at[c, pl.ds(t * rpa, rpa), :])


@functools.partial(
    pl.kernel,
    mesh=_mesh,
    compiler_params=_cparams,
    out_type=jax.ShapeDtypeStruct((2, _HPAD, 128), jnp.float32),
    scratch_types=[
        pltpu.VMEM((16,), jnp.float32),
        pltpu.VMEM((10000,), jnp.int32),
        pltpu.VMEM((10000,), jnp.int32),
        pltpu.VMEM((10000,), jnp.float32),
        pltpu.VMEM((10000,), jnp.float32),
        pltpu.VMEM((_CH,), jnp.int32),
        pltpu.VMEM((_CH,), jnp.int32),
        pltpu.VMEM((_CH,), jnp.int32),
        pltpu.VMEM((_CH,), jnp.int32),
        pltpu.VMEM((_CH, 128), jnp.float32),
        pltpu.VMEM((_CH, 128), jnp.float32),
        pltpu.VMEM((_CH,), jnp.float32),
        pltpu.VMEM((_CH, 128), jnp.float32),
        pltpu.VMEM((_CH, 128), jnp.float32),
        pltpu.VMEM_SHARED((_HPAD, 128), jnp.float32),
        pltpu.SemaphoreType.DMA,
        pltpu.SemaphoreType.DMA,
        pltpu.SemaphoreType.DMA,
        pltpu.SemaphoreType.DMA,
    ],
)
def _sc_pass2b(src, dst, ea0, ea1, h1, prmb, outb,
               prm_v, src_v, dst_v, ea0_v, ea1_v,
               srcca, srccb, dstc2a, dstc2b, rowsa, rowsb, gbuf,
               pbufa, pbufb, slabb, sem_ra, sem_rb, sem_sa, sem_sb):
    c = lax.axis_index("c")
    t = lax.axis_index("s")
    zeros16 = jnp.zeros((16,), jnp.float32)

    pltpu.sync_copy(prmb, prm_v)

    def _z(i, _):
        for jr in range(8):
            pbufa[i, pl.ds(16 * jr, 16)] = zeros16
            pbufb[i, pl.ds(16 * jr, 16)] = zeros16
        return 0
    lax.fori_loop(0, _CH, _z, 0)
    rpb = _HPAD // 16
    for j in range(rpb // (2 * _CH)):
        pltpu.sync_copy(pbufa, slabb.at[pl.ds(t * rpb + 2 * j * _CH, _CH), :])
        pltpu.sync_copy(pbufb,
                        slabb.at[pl.ds(t * rpb + (2 * j + 1) * _CH, _CH), :])
    plsc.subcore_barrier()

    pv = prm_v[...]
    pv_c = -0.5 / (_EPS + pv * pv)
    m0 = pv[0]
    m1 = pv[1]
    c0 = pv_c[2]
    c1 = pv_c[3]

    base = pl.multiple_of((c * 16 + t) * 10000, 8)
    pltpu.sync_copy(src.at[pl.ds(base, 10000)], src_v)
    pltpu.sync_copy(dst.at[pl.ds(base, 10000)], dst_v)
    pltpu.sync_copy(ea0.at[pl.ds(base, 10000)], ea0_v)
    pltpu.sync_copy(ea1.at[pl.ds(base, 10000)], ea1_v)

    def _fill_src(buf, loc):
        for g in range(5):
            buf[pl.ds(16 * g, 16)] = src_v[pl.ds(loc + 16 * g, 16)]

    def _fill_dst2(buf, loc):
        for g in range(5):
            d16 = dst_v[pl.ds(loc + 16 * g, 16)]
            buf[pl.ds(16 * g, 16)] = lax.shift_right_logical(d16, 1)

    def _gauss(loc):
        for g in range(5):
            e0 = ea0_v[pl.ds(loc + 16 * g, 16)]
            e1 = ea1_v[pl.ds(loc + 16 * g, 16)]
            d0 = e0 - m0
            d1 = e1 - m1
            gbuf[pl.ds(16 * g, 16)] = jnp.exp(d0 * d0 * c0 + d1 * d1 * c1)

    def _payload(loc, rows, pbuf):
        def _edge4(ii, _):
            for u in range(4):
                e = 4 * ii + u
                s2 = plsc.load_gather(gbuf, [jnp.full((16,), e, jnp.int32)])
                dv = plsc.load_gather(dst_v,
                                      [jnp.full((16,), loc + e, jnp.int32)])
                even = (dv & 1) == 0
                for jr in range(4):
                    v = rows[e, pl.ds(16 * jr, 16)] * s2
                    pbuf[e, pl.ds(16 * jr, 16)] = jnp.where(even, v, zeros16)
                    pbuf[e, pl.ds(64 + 16 * jr, 16)] = \
                        jnp.where(even, zeros16, v)
            return 0
        lax.fori_loop(0, _CH // 4, _edge4, 0)

    _fill_src(srcca, 0)
    pltpu.async_copy(h1.at[srcca], rowsa, sem_ra)

    def _body(i, _):
        loc0 = 2 * i * _CH
        loc1 = loc0 + _CH

        pltpu.make_async_copy(h1.at[srcca], rowsa, sem_ra).wait()
        _gauss(loc0)
        _fill_src(srccb, loc1)
        pltpu.async_copy(h1.at[srccb], rowsb, sem_rb)

        @pl.when(i > 0)
        def _w_sa():
            pltpu.make_async_copy(pbufa, slabb.at[dstc2a], sem_sa).wait()
        _fill_dst2(dstc2a, loc0)
        _payload(loc0, rowsa, pbufa)
        pltpu.async_copy(pbufa, slabb.at[dstc2a], sem_sa, add=True)

        pltpu.make_async_copy(h1.at[srccb], rowsb, sem_rb).wait()
        _gauss(loc1)

        @pl.when(i > 0)
        def _w_sb():
            pltpu.make_async_copy(pbufb, slabb.at[dstc2b], sem_sb).wait()
        _fill_dst2(dstc2b, loc1)
        _fill_src(srcca, loc1 + _CH)
        pltpu.async_copy(h1.at[srcca], rowsa, sem_ra)
        _payload(loc1, rowsb, pbufb)
        pltpu.async_copy(pbufb, slabb.at[dstc2b], sem_sb, add=True)
        return 0

    lax.fori_loop(0, 62, _body, 0)

    loct = 124 * _CH
    pltpu.make_async_copy(h1.at[srcca], rowsa, sem_ra).wait()
    _gauss(loct)
    pltpu.make_async_copy(pbufa, slabb.at[dstc2a], sem_sa).wait()
    _fill_dst2(dstc2a, loct)
    _payload(loct, rowsa, pbufa)
    pltpu.async_copy(pbufa, slabb.at[dstc2a], sem_sa, add=True)

    pltpu.make_async_copy(pbufa, slabb.at[dstc2a], sem_sa).wait()
    pltpu.make_async_copy(pbufb, slabb.at[dstc2b], sem_sb).wait()
    plsc.subcore_barrier()
    pltpu.sync_copy(slabb.at[pl.ds(t * rpb, rpb), :],
                    outb.at[c, pl.ds(t * rpb, rpb), :])


def _d1_body(s1_ref, x_ref, g1_ref, root1_ref, b1_ref, h1_ref, inv_ref):
    s = s1_ref[0] + s1_ref[1]
    a = s[:_N, 0:5]
    cnt = s[:_N, 5:6]
    inv = 1.0 / jnp.maximum(cnt, 1.0)
    h = jnp.dot(a, g1_ref[...], preferred_element_type=jnp.float32) * inv
    h = h + jnp.dot(x_ref[...], root1_ref[...],
                    preferred_element_type=jnp.float32) + b1_ref[...][None, :]
    h = jnp.maximum(h, 0.0)
    h1_ref[...] = jnp.concatenate(
        [h, jnp.zeros((_N, 64), jnp.float32)], axis=1)
    inv_ref[...] = inv


def _dense1(slab1r, x, G1, root1, b1):
    return pl.pallas_call(
        _d1_body,
        out_shape=[jax.ShapeDtypeStruct((_N, 128), jnp.float32),
                   jax.ShapeDtypeStruct((_N, 1), jnp.float32)],
    )(slab1r, x, G1, root1, b1)


def _d2_body(sa_ref, sb_ref, h1_ref, inv_ref, g2_ref, root2_ref, b2_ref,
             batch_ref, wf1_ref, bf1_ref, wf2_ref, bf2_ref, out_ref):
    k2 = sb_ref[0] + sb_ref[1]
    acat = jnp.concatenate([
        sa_ref[0][:_N, 0:64], sa_ref[0][:_N, 64:128],
        k2[:_N, :],
        sa_ref[1][:_N, 0:64], sa_ref[1][:_N, 64:128]], axis=1)
    agg = jnp.dot(acat, g2_ref[...],
                  preferred_element_type=jnp.float32) * inv_ref[...]
    h2 = agg + jnp.dot(h1_ref[...][:, 0:64], root2_ref[...],
                       preferred_element_type=jnp.float32) + b2_ref[...][None, :]
    h2 = jnp.maximum(h2, 0.0)
    gid = lax.broadcasted_iota(jnp.int32, (_NG, 1), 0)
    pm = (batch_ref[...] == gid).astype(jnp.float32)
    cg = jnp.sum(pm, axis=1, keepdims=True)
    p = jnp.dot(pm, h2, preferred_element_type=jnp.float32) / jnp.maximum(cg, 1.0)
    p = jnp.maximum(jnp.dot(p, wf1_ref[...], preferred_element_type=jnp.float32)
                    + bf1_ref[...][None, :], 0.0)
    lo = jnp.dot(p, wf2_ref[...], preferred_element_type=jnp.float32) \
        + bf2_ref[...][None, :]
    m = jnp.max(lo, axis=1, keepdims=True)
    lse = jnp.log(jnp.sum(jnp.exp(lo - m), axis=1, keepdims=True)) + m
    out_ref[...] = lo - lse


def _dense2(slaba, slabbr, h1, inv, G2cat, root2, b2, batch2d,
            Wf1, bf1, Wf2, bf2):
    return pl.pallas_call(
        _d2_body,
        out_shape=jax.ShapeDtypeStruct((_NG, 10), jnp.float32),
    )(slaba, slabbr, h1, inv, G2cat, root2, b2, batch2d, Wf1, bf1, Wf2, bf2)


def kernel(x, edge_index, edge_attr, batch, g1, mu1, sigma1, root1, b1,
           g2, mu2, sigma2, root2, b2, Wf1, bf1, Wf2, bf2):
    x1d = x.reshape(_N)
    batch2d = batch.reshape(1, _N)
    G1 = g1.reshape(_K, 64)
    G2cat = g2.reshape(64, _K, 128).transpose(1, 0, 2).reshape(_K * 64, 128)
    padz = jnp.zeros((11,), jnp.float32)
    pado = jnp.ones((11,), jnp.float32)
    pad10 = jnp.zeros((10,), jnp.float32)
    prm1 = jnp.stack([
        jnp.concatenate([mu1[:, 0], padz]),
        jnp.concatenate([mu1[:, 1], padz]),
        jnp.concatenate([sigma1[:, 0], pado]),
        jnp.concatenate([sigma1[:, 1], pado])])
    pad12z = jnp.zeros((12,), jnp.float32)
    pad12o = jnp.ones((12,), jnp.float32)
    sel0 = jnp.array([0, 1], dtype=jnp.int32)
    sel1 = jnp.array([3, 4], dtype=jnp.int32)
    prm2 = jnp.stack([
        jnp.concatenate([mu2[sel0].reshape(-1), pad12z,
                         sigma2[sel0].reshape(-1), pad12o]),
        jnp.concatenate([mu2[sel1].reshape(-1), pad12z,
                         sigma2[sel1].reshape(-1), pad12o])])
    prmb = jnp.concatenate([mu2[2], sigma2[2], jnp.ones((12,), jnp.float32)])
    src = edge_index[0]
    dst = edge_index[1]
    ea0 = edge_attr[:, 0]
    ea1 = edge_attr[:, 1]

    slab1 = _sc_pass1(src, dst, ea0, ea1, x1d, prm1)
    slab1r = slab1.reshape(2, _NPAD, 64)
    h1, inv = _dense1(slab1r, x, G1, root1, b1)
    slaba = _sc_pass2a(src, dst, ea0, ea1, h1, prm2)
    slabb = _sc_pass2b(src, dst, ea0, ea1, h1, prmb)
    slabbr = slabb.reshape(2, _NPAD, 64)
    return _dense2(slaba, slabbr, h1, inv, G2cat, root2, b2, batch2d,
                   Wf1, bf1, Wf2, bf2)


# speedup vs baseline: 9.2588x; 1.1013x over previous
import functools

import jax
import jax.numpy as jnp
from jax import lax
from jax.experimental import pallas as pl
from jax.experimental.pallas import tpu as pltpu
from jax.experimental.pallas import tpu_sc as plsc

_N = 10000
_E = 320000
_K = 5
_NG = 64
_EPS = 1e-15
_NPAD = 10240
_HPAD = 5120
_CH = 80

_mesh = plsc.VectorSubcoreMesh(core_axis_name="c", subcore_axis_name="s")
_cparams = pltpu.CompilerParams(needs_layout_passes=False)


@functools.partial(
    pl.kernel,
    mesh=_mesh,
    compiler_params=_cparams,
    out_type=jax.ShapeDtypeStruct((2, _HPAD, 128), jnp.float32),
    scratch_types=[
        pltpu.VMEM((_N,), jnp.float32),
        pltpu.VMEM((4, 16), jnp.float32),
        pltpu.VMEM((10000,), jnp.int32),
        pltpu.VMEM((10000,), jnp.int32),
        pltpu.VMEM((10000,), jnp.float32),
        pltpu.VMEM((10000,), jnp.float32),
        pltpu.VMEM((_CH,), jnp.int32),
        pltpu.VMEM((_CH,), jnp.int32),
        pltpu.VMEM((416,), jnp.float32),
        pltpu.VMEM((_CH, 128), jnp.float32),
        pltpu.VMEM((_CH, 128), jnp.float32),
        pltpu.VMEM_SHARED((_HPAD, 128), jnp.float32),
        pltpu.SemaphoreType.DMA,
        pltpu.SemaphoreType.DMA,
    ],
)
def _sc_pass1(src, dst, ea0, ea1, x, prm, out,
              x_v, prm_v, src_v, dst_v, ea0_v, ea1_v, dstc2a, dstc2b,
              pgbuf, pbufa, pbufb, slab, sem_sa, sem_sb):
    c = lax.axis_index("c")
    t = lax.axis_index("s")
    lane = lax.iota(jnp.int32, 16)
    zeros16 = jnp.zeros((16,), jnp.float32)

    pltpu.sync_copy(x, x_v)
    pltpu.sync_copy(prm, prm_v)
    base = pl.multiple_of((c * 16 + t) * 10000, 8)
    pltpu.sync_copy(src.at[pl.ds(base, 10000)], src_v)
    pltpu.sync_copy(dst.at[pl.ds(base, 10000)], dst_v)
    pltpu.sync_copy(ea0.at[pl.ds(base, 10000)], ea0_v)
    pltpu.sync_copy(ea1.at[pl.ds(base, 10000)], ea1_v)

    def _z(i, _):
        for jr in range(8):
            pbufa[i, pl.ds(16 * jr, 16)] = zeros16
            pbufb[i, pl.ds(16 * jr, 16)] = zeros16
        return 0
    lax.fori_loop(0, _CH, _z, 0)
    rpt = _HPAD // 16
    for j in range(rpt // (2 * _CH)):
        pltpu.sync_copy(pbufa, slab.at[pl.ds(t * rpt + 2 * j * _CH, _CH), :])
        pltpu.sync_copy(pbufb,
                        slab.at[pl.ds(t * rpt + (2 * j + 1) * _CH, _CH), :])
    plsc.subcore_barrier()

    mv0 = prm_v[0, :]
    mv1 = prm_v[1, :]
    sv0 = prm_v[2, :]
    sv1 = prm_v[3, :]
    cv0 = -0.5 / (_EPS + sv0 * sv0)
    cv1 = -0.5 / (_EPS + sv1 * sv1)
    is_g = lane < 5
    mk = [(mv0[k], mv1[k]) for k in range(5)]
    ck = [(cv0[k], cv1[k]) for k in range(5)]

    pgbuf[pl.ds(400, 16)] = jnp.where(lane == 0, 1.0, 0.0)
    ibase = jnp.where(is_g, lane * 80, jnp.where(lane == 5, 400, 408))

    def _fill_dst2(buf, loc):
        for g in range(5):
            d16 = dst_v[pl.ds(loc + 16 * g, 16)]
            buf[pl.ds(16 * g, 16)] = lax.shift_right_logical(d16, 1)

    def _payload(loc, pbuf):
        for g in range(5):
            src16 = src_v[pl.ds(loc + 16 * g, 16)]
            xs = plsc.load_gather(x_v, [src16])
            e0 = ea0_v[pl.ds(loc + 16 * g, 16)]
            e1 = ea1_v[pl.ds(loc + 16 * g, 16)]
            for k in range(5):
                d0 = e0 - mk[k][0]
                d1 = e1 - mk[k][1]
                gv = jnp.exp(d0 * d0 * ck[k][0] + d1 * d1 * ck[k][1])
                pgbuf[pl.ds(80 * k + 16 * g, 16)] = gv * xs

        def _edge4(ii, _2):
            for u in range(4):
                e = 4 * ii + u
                ef = jnp.full((16,), e, jnp.int32)
                p16 = plsc.load_gather(pgbuf,
                                       [ibase + jnp.where(is_g, ef, 0)])
                dv = plsc.load_gather(dst_v,
                                      [jnp.full((16,), loc + e, jnp.int32)])
                even = (dv & 1) == 0
                pbuf[e, pl.ds(0, 16)] = jnp.where(even, p16, zeros16)
                pbuf[e, pl.ds(64, 16)] = jnp.where(even, zeros16, p16)
            return 0
        lax.fori_loop(0, _CH // 4, _edge4, 0)

    def _body(i, _):
        loc0 = 2 * i * _CH
        loc1 = loc0 + _CH

        @pl.when(i > 0)
        def _w_sa():
            pltpu.make_async_copy(pbufa, slab.at[dstc2a], sem_sa).wait()
        _fill_dst2(dstc2a, loc0)
        _payload(loc0, pbufa)
        pltpu.async_copy(pbufa, slab.at[dstc2a], sem_sa, add=True)

        @pl.when(i > 0)
        def _w_sb():
            pltpu.make_async_copy(pbufb, slab.at[dstc2b], sem_sb).wait()
        _fill_dst2(dstc2b, loc1)
        _payload(loc1, pbufb)
        pltpu.async_copy(pbufb, slab.at[dstc2b], sem_sb, add=True)
        return 0

    lax.fori_loop(0, 62, _body, 0)

    loct = 124 * _CH
    pltpu.make_async_copy(pbufa, slab.at[dstc2a], sem_sa).wait()
    _fill_dst2(dstc2a, loct)
    _payload(loct, pbufa)
    pltpu.async_copy(pbufa, slab.at[dstc2a], sem_sa, add=True)

    pltpu.make_async_copy(pbufa, slab.at[dstc2a], sem_sa).wait()
    pltpu.make_async_copy(pbufb, slab.at[dstc2b], sem_sb).wait()
    plsc.subcore_barrier()
    pltpu.sync_copy(slab.at[pl.ds(t * rpt, rpt), :],
                    out.at[c, pl.ds(t * rpt, rpt), :])


@functools.partial(
    pl.kernel,
    mesh=_mesh,
    compiler_params=_cparams,
    out_type=jax.ShapeDtypeStruct((2, _NPAD, 128), jnp.float32),
    scratch_types=[
        pltpu.VMEM((2, 32), jnp.float32),
        pltpu.VMEM((800,), jnp.int32),
        pltpu.VMEM((800,), jnp.int32),
        pltpu.VMEM((800,), jnp.float32),
        pltpu.VMEM((800,), jnp.float32),
        pltpu.VMEM((_CH,), jnp.int32),
        pltpu.VMEM((_CH,), jnp.int32),
        pltpu.VMEM((_CH,), jnp.int32),
        pltpu.VMEM((_CH,), jnp.int32),
        pltpu.VMEM((_CH, 128), jnp.float32),
        pltpu.VMEM((_CH, 128), jnp.float32),
        pltpu.VMEM((2 * _CH,), jnp.float32),
        pltpu.VMEM((_CH, 128), jnp.float32),
        pltpu.VMEM((_CH, 128), jnp.float32),
        pltpu.VMEM_SHARED((_NPAD, 128), jnp.float32),
        pltpu.SemaphoreType.DMA,
        pltpu.SemaphoreType.DMA,
        pltpu.SemaphoreType.DMA,
        pltpu.SemaphoreType.DMA,
    ],
)
def _sc_pass2a(src, dst, ea0, ea1, h1, prm2, outa,
               prm_v, src_sv, dst_sv, ea0_sv, ea1_sv,
               srcca, srccb, dstca, dstcb, rowsa, rowsb, gbuf,
               pbufa, pbufb, slaba, sem_ra, sem_rb, sem_sa, sem_sb):
    c = lax.axis_index("c")
    t = lax.axis_index("s")
    zeros16 = jnp.zeros((16,), jnp.float32)

    pltpu.sync_copy(prm2, prm_v)

    def _z(i, _):
        for jr in range(8):
            pbufa[i, pl.ds(16 * jr, 16)] = zeros16
            pbufb[i, pl.ds(16 * jr, 16)] = zeros16
        return 0
    lax.fori_loop(0, _CH, _z, 0)
    rpa = _NPAD // 16
    for j in range(rpa // (2 * _CH)):
        pltpu.sync_copy(pbufa, slaba.at[pl.ds(t * rpa + 2 * j * _CH, _CH), :])
        pltpu.sync_copy(pbufb,
                        slaba.at[pl.ds(t * rpa + (2 * j + 1) * _CH, _CH), :])
    plsc.subcore_barrier()

    pv_m = prm_v[c, pl.ds(0, 16)]
    pv_s = prm_v[c, pl.ds(16, 16)]
    pv_c = -0.5 / (_EPS + pv_s * pv_s)
    mks, cks = [], []
    for kk in range(2):
        mks.append((pv_m[2 * kk], pv_m[2 * kk + 1]))
        cks.append((pv_c[2 * kk], pv_c[2 * kk + 1]))

    tbase = t * 20000

    def _load_super(s_idx):
        sb = pl.multiple_of(tbase + s_idx * 800, 8)
        pltpu.sync_copy(src.at[pl.ds(sb, 800)], src_sv)
        pltpu.sync_copy(dst.at[pl.ds(sb, 800)], dst_sv)
        pltpu.sync_copy(ea0.at[pl.ds(sb, 800)], ea0_sv)
        pltpu.sync_copy(ea1.at[pl.ds(sb, 800)], ea1_sv)

    def _fill_idx(buf, sv, loc):
        for g in range(5):
            buf[pl.ds(16 * g, 16)] = sv[pl.ds(loc + 16 * g, 16)]

    def _gauss(loc):
        for g in range(5):
            e0 = ea0_sv[pl.ds(loc + 16 * g, 16)]
            e1 = ea1_sv[pl.ds(loc + 16 * g, 16)]
            for kk in range(2):
                d0 = e0 - mks[kk][0]
                d1 = e1 - mks[kk][1]
                gk = jnp.exp(d0 * d0 * cks[kk][0] + d1 * d1 * cks[kk][1])
                gbuf[pl.ds(80 * kk + 16 * g, 16)] = gk

    def _payload(rows, pbuf):
        def _edge4(ii, _):
            for u in range(4):
                e = 4 * ii + u
                s0 = plsc.load_gather(gbuf, [jnp.full((16,), e, jnp.int32)])
                s1 = plsc.load_gather(gbuf,
                                      [jnp.full((16,), 80 + e, jnp.int32)])
                for jr in range(4):
                    r = rows[e, pl.ds(16 * jr, 16)]
                    pbuf[e, pl.ds(16 * jr, 16)] = r * s0
                    pbuf[e, pl.ds(64 + 16 * jr, 16)] = r * s1
            return 0
        lax.fori_loop(0, _CH // 4, _edge4, 0)

    _load_super(0)
    _fill_idx(srcca, src_sv, 0)
    pltpu.async_copy(h1.at[srcca], rowsa, sem_ra)

    def _body(i, _):
        m0 = 2 * i
        m1 = 2 * i + 1
        loc0 = lax.rem(m0, 10) * _CH
        loc1 = lax.rem(m1, 10) * _CH

        pltpu.make_async_copy(h1.at[srcca], rowsa, sem_ra).wait()
        _gauss(loc0)
        _fill_idx(srccb, src_sv, loc1)
        pltpu.async_copy(h1.at[srccb], rowsb, sem_rb)

        @pl.when(i > 0)
        def _w_sa():
            pltpu.make_async_copy(pbufa, slaba.at[dstca], sem_sa).wait()
        _fill_idx(dstca, dst_sv, loc0)
        _payload(rowsa, pbufa)
        pltpu.async_copy(pbufa, slaba.at[dstca], sem_sa, add=True)

        pltpu.make_async_copy(h1.at[srccb], rowsb, sem_rb).wait()
        _gauss(loc1)

        @pl.when(i > 0)
        def _w_sb():
            pltpu.make_async_copy(pbufb, slaba.at[dstcb], sem_sb).wait()
        _fill_idx(dstcb, dst_sv, loc1)

        @pl.when(jnp.logical_and(lax.rem(i, 5) == 4, i < 124))
        def _ns():
            _load_super((m1 + 1) // 10)

        @pl.when(i < 124)
        def _pf():
            loc2 = lax.rem(m1 + 1, 10) * _CH
            _fill_idx(srcca, src_sv, loc2)
            pltpu.async_copy(h1.at[srcca], rowsa, sem_ra)

        _payload(rowsb, pbufb)
        pltpu.async_copy(pbufb, slaba.at[dstcb], sem_sb, add=True)
        return 0

    lax.fori_loop(0, 125, _body, 0)
    pltpu.make_async_copy(pbufa, slaba.at[dstca], sem_sa).wait()
    pltpu.make_async_copy(pbufb, slaba.at[dstcb], sem_sb).wait()
    plsc.subcore_barrier()
    pltpu.sync_copy(slaba.at[pl.ds(t * rpa, rpa), :],
                    outa.at[c, pl.ds(t * rpa, rpa), :])


@functools.partial(
    pl.kernel,
    mesh=_mesh,
    compiler_params=_cparams,
    out_type=jax.ShapeDtypeStruct((2, _HPAD, 128), jnp.float32),
    scratch_types=[
        pltpu.VMEM((16,), jnp.float32),
        pltpu.VMEM((10000,), jnp.int32),
        pltpu.VMEM((10000,), jnp.int32),
        pltpu.VMEM((10000,), jnp.float32),
        pltpu.VMEM((10000,), jnp.float32),
        pltpu.VMEM((_CH,), jnp.int32),
        pltpu.VMEM((_CH,), jnp.int32),
        pltpu.VMEM((_CH,), jnp.int32),
        pltpu.VMEM((_CH,), jnp.int32),
        pltpu.VMEM((_CH, 128), jnp.float32),
        pltpu.VMEM((_CH, 128), jnp.float32),
        pltpu.VMEM((_CH,), jnp.float32),
        pltpu.VMEM((_CH, 128), jnp.float32),
        pltpu.VMEM((_CH, 128), jnp.float32),
        pltpu.VMEM_SHARED((_HPAD, 128), jnp.float32),
        pltpu.SemaphoreType.DMA,
        pltpu.SemaphoreType.DMA,
        pltpu.SemaphoreType.DMA,
        pltpu.SemaphoreType.DMA,
    ],
)
def _sc_pass2b(src, dst, ea0, ea1, h1, prmb, outb,
               prm_v, src_v, dst_v, ea0_v, ea1_v,
               srcca, srccb, dstc2a, dstc2b, rowsa, rowsb, gbuf,
               pbufa, pbufb, slabb, sem_ra, sem_rb, sem_sa, sem_sb):
    c = lax.axis_index("c")
    t = lax.axis_index("s")
    zeros16 = jnp.zeros((16,), jnp.float32)

    pltpu.sync_copy(prmb, prm_v)

    def _z(i, _):
        for jr in range(8):
            pbufa[i, pl.ds(16 * jr, 16)] = zeros16
            pbufb[i, pl.ds(16 * jr, 16)] = zeros16
        return 0
    lax.fori_loop(0, _CH, _z, 0)
    rpb = _HPAD // 16
    for j in range(rpb // (2 * _CH)):
        pltpu.sync_copy(pbufa, slabb.at[pl.ds(t * rpb + 2 * j * _CH, _CH), :])
        pltpu.sync_copy(pbufb,
                        slabb.at[pl.ds(t * rpb + (2 * j + 1) * _CH, _CH), :])
    plsc.subcore_barrier()

    pv = prm_v[...]
    pv_c = -0.5 / (_EPS + pv * pv)
    m0 = pv[0]
    m1 = pv[1]
    c0 = pv_c[2]
    c1 = pv_c[3]

    base = pl.multiple_of((c * 16 + t) * 10000, 8)
    pltpu.sync_copy(src.at[pl.ds(base, 10000)], src_v)
    pltpu.sync_copy(dst.at[pl.ds(base, 10000)], dst_v)
    pltpu.sync_copy(ea0.at[pl.ds(base, 10000)], ea0_v)
    pltpu.sync_copy(ea1.at[pl.ds(base, 10000)], ea1_v)

    def _fill_src(buf, loc):
        for g in range(5):
            buf[pl.ds(16 * g, 16)] = src_v[pl.ds(loc + 16 * g, 16)]

    def _fill_dst2(buf, loc):
        for g in range(5):
            d16 = dst_v[pl.ds(loc + 16 * g, 16)]
            buf[pl.ds(16 * g, 16)] = lax.shift_right_logical(d16, 1)

    def _gauss(loc):
        for g in range(5):
            e0 = ea0_v[pl.ds(loc + 16 * g, 16)]
            e1 = ea1_v[pl.ds(loc + 16 * g, 16)]
            d0 = e0 - m0
            d1 = e1 - m1
            gbuf[pl.ds(16 * g, 16)] = jnp.exp(d0 * d0 * c0 + d1 * d1 * c1)

    def _payload(loc, rows, pbuf):
        def _edge4(ii, _):
            for u in range(4):
                e = 4 * ii + u
                s2 = plsc.load_gather(gbuf, [jnp.full((16,), e, jnp.int32)])
                dv = plsc.load_gather(dst_v,
                                      [jnp.full((16,), loc + e, jnp.int32)])
                even = (dv & 1) == 0
                for jr in range(4):
                    v = rows[e, pl.ds(16 * jr, 16)] * s2
                    pbuf[e, pl.ds(16 * jr, 16)] = jnp.where(even, v, zeros16)
                    pbuf[e, pl.ds(64 + 16 * jr, 16)] = \
                        jnp.where(even, zeros16, v)
            return 0
        lax.fori_loop(0, _CH // 4, _edge4, 0)

    _fill_src(srcca, 0)
    pltpu.async_copy(h1.at[srcca], rowsa, sem_ra)

    def _body(i, _):
        loc0 = 2 * i * _CH
        loc1 = loc0 + _CH

        pltpu.make_async_copy(h1.at[srcca], rowsa, sem_ra).wait()
        _gauss(loc0)
        _fill_src(srccb, loc1)
        pltpu.async_copy(h1.at[srccb], rowsb, sem_rb)

        @pl.when(i > 0)
        def _w_sa():
            pltpu.make_async_copy(pbufa, slabb.at[dstc2a], sem_sa).wait()
        _fill_dst2(dstc2a, loc0)
        _payload(loc0, rowsa, pbufa)
        pltpu.async_copy(pbufa, slabb.at[dstc2a], sem_sa, add=True)

        pltpu.make_async_copy(h1.at[srccb], rowsb, sem_rb).wait()
        _gauss(loc1)

        @pl.when(i > 0)
        def _w_sb():
            pltpu.make_async_copy(pbufb, slabb.at[dstc2b], sem_sb).wait()
        _fill_dst2(dstc2b, loc1)
        _fill_src(srcca, loc1 + _CH)
        pltpu.async_copy(h1.at[srcca], rowsa, sem_ra)
        _payload(loc1, rowsb, pbufb)
        pltpu.async_copy(pbufb, slabb.at[dstc2b], sem_sb, add=True)
        return 0

    lax.fori_loop(0, 62, _body, 0)

    loct = 124 * _CH
    pltpu.make_async_copy(h1.at[srcca], rowsa, sem_ra).wait()
    _gauss(loct)
    pltpu.make_async_copy(pbufa, slabb.at[dstc2a], sem_sa).wait()
    _fill_dst2(dstc2a, loct)
    _payload(loct, rowsa, pbufa)
    pltpu.async_copy(pbufa, slabb.at[dstc2a], sem_sa, add=True)

    pltpu.make_async_copy(pbufa, slabb.at[dstc2a], sem_sa).wait()
    pltpu.make_async_copy(pbufb, slabb.at[dstc2b], sem_sb).wait()
    plsc.subcore_barrier()
    pltpu.sync_copy(slabb.at[pl.ds(t * rpb, rpb), :],
                    outb.at[c, pl.ds(t * rpb, rpb), :])


def _d1_body(s1_ref, x_ref, g1_ref, root1_ref, b1_ref, h1_ref, inv_ref):
    s = s1_ref[0] + s1_ref[1]
    a = s[:_N, 0:5]
    cnt = s[:_N, 5:6]
    inv = 1.0 / jnp.maximum(cnt, 1.0)
    h = jnp.dot(a, g1_ref[...], preferred_element_type=jnp.float32) * inv
    h = h + jnp.dot(x_ref[...], root1_ref[...],
                    preferred_element_type=jnp.float32) + b1_ref[...][None, :]
    h = jnp.maximum(h, 0.0)
    h1_ref[...] = jnp.concatenate(
        [h, jnp.zeros((_N, 64), jnp.float32)], axis=1)
    inv_ref[...] = inv


def _dense1(slab1r, x, G1, root1, b1):
    return pl.pallas_call(
        _d1_body,
        out_shape=[jax.ShapeDtypeStruct((_N, 128), jnp.float32),
                   jax.ShapeDtypeStruct((_N, 1), jnp.float32)],
    )(slab1r, x, G1, root1, b1)


def _d2_body(sa_ref, sb_ref, h1_ref, inv_ref, g2_ref, root2_ref, b2_ref,
             batch_ref, wf1_ref, bf1_ref, wf2_ref, bf2_ref, out_ref):
    k2 = sb_ref[0] + sb_ref[1]
    acat = jnp.concatenate([
        sa_ref[0][:_N, 0:64], sa_ref[0][:_N, 64:128],
        k2[:_N, :],
        sa_ref[1][:_N, 0:64], sa_ref[1][:_N, 64:128]], axis=1)
    agg = jnp.dot(acat, g2_ref[...],
                  preferred_element_type=jnp.float32) * inv_ref[...]
    h2 = agg + jnp.dot(h1_ref[...][:, 0:64], root2_ref[...],
                       preferred_element_type=jnp.float32) + b2_ref[...][None, :]
    h2 = jnp.maximum(h2, 0.0)
    gid = lax.broadcasted_iota(jnp.int32, (_NG, 1), 0)
    pm = (batch_ref[...] == gid).astype(jnp.float32)
    cg = jnp.sum(pm, axis=1, keepdims=True)
    p = jnp.dot(pm, h2, preferred_element_type=jnp.float32) / jnp.maximum(cg, 1.0)
    p = jnp.maximum(jnp.dot(p, wf1_ref[...], preferred_element_type=jnp.float32)
                    + bf1_ref[...][None, :], 0.0)
    lo = jnp.dot(p, wf2_ref[...], preferred_element_type=jnp.float32) \
        + bf2_ref[...][None, :]
    m = jnp.max(lo, axis=1, keepdims=True)
    lse = jnp.log(jnp.sum(jnp.exp(lo - m), axis=1, keepdims=True)) + m
    out_ref[...] = lo - lse


def _dense2(slaba, slabbr, h1, inv, G2cat, root2, b2, batch2d,
            Wf1, bf1, Wf2, bf2):
    return pl.pallas_call(
        _d2_body,
        out_shape=jax.ShapeDtypeStruct((_NG, 10), jnp.float32),
    )(slaba, slabbr, h1, inv, G2cat, root2, b2, batch2d, Wf1, bf1, Wf2, bf2)


def kernel(x, edge_index, edge_attr, batch, g1, mu1, sigma1, root1, b1,
           g2, mu2, sigma2, root2, b2, Wf1, bf1, Wf2, bf2):
    x1d = x.reshape(_N)
    batch2d = batch.reshape(1, _N)
    G1 = g1.reshape(_K, 64)
    G2cat = g2.reshape(64, _K, 128).transpose(1, 0, 2).reshape(_K * 64, 128)
    padz = jnp.zeros((11,), jnp.float32)
    pado = jnp.ones((11,), jnp.float32)
    pad10 = jnp.zeros((10,), jnp.float32)
    prm1 = jnp.stack([
        jnp.concatenate([mu1[:, 0], padz]),
        jnp.concatenate([mu1[:, 1], padz]),
        jnp.concatenate([sigma1[:, 0], pado]),
        jnp.concatenate([sigma1[:, 1], pado])])
    pad12z = jnp.zeros((12,), jnp.float32)
    pad12o = jnp.ones((12,), jnp.float32)
    sel0 = jnp.array([0, 1], dtype=jnp.int32)
    sel1 = jnp.array([3, 4], dtype=jnp.int32)
    prm2 = jnp.stack([
        jnp.concatenate([mu2[sel0].reshape(-1), pad12z,
                         sigma2[sel0].reshape(-1), pad12o]),
        jnp.concatenate([mu2[sel1].reshape(-1), pad12z,
                         sigma2[sel1].reshape(-1), pad12o])])
    prmb = jnp.concatenate([mu2[2], sigma2[2], jnp.ones((12,), jnp.float32)])
    src = edge_index[0]
    dst = edge_index[1]
    ea0 = edge_attr[:, 0]
    ea1 = edge_attr[:, 1]

    slab1 = _sc_pass1(src, dst, ea0, ea1, x1d, prm1)
    slab1r = slab1.reshape(2, _NPAD, 64)
    h1, inv = _dense1(slab1r, x, G1, root1, b1)
    slaba = _sc_pass2a(src, dst, ea0, ea1, h1, prm2)
    slabb = _sc_pass2b(src, dst, ea0, ea1, h1, prmb)
    slabbr = slabb.reshape(2, _NPAD, 64)
    return _dense2(slaba, slabbr, h1, inv, G2cat, root2, b2, batch2d,
                   Wf1, bf1, Wf2, bf2)
